# baseline (speedup 1.0000x reference)
import functools

import jax
import jax.numpy as jnp
from jax import lax
from jax.experimental import pallas as pl
from jax.experimental.pallas import tpu as pltpu
from jax.experimental.pallas import tpu_sc as plsc

N = 10000
NPAD = 10240
D_IN = 128
DH = 64
E = 320000
C = 400
TILES = 16
EPT = E // TILES
NCHUNK = EPT // C
STRIPE = NPAD // TILES
OSTRIPE = N // TILES


def _mm_body(x_ref, w_ref, o_ref):
    h = jnp.dot(x_ref[...], w_ref[...], preferred_element_type=jnp.float32)
    o_ref[0, :N] = h[:, :DH]
    o_ref[1, :N] = h[:, DH:]


_matmul = pl.pallas_call(
    _mm_body,
    out_shape=jax.ShapeDtypeStruct((2, NPAD, DH), jnp.float32),
)

_sc_mesh = plsc.VectorSubcoreMesh(core_axis_name="c", subcore_axis_name="s")


@functools.partial(
    pl.kernel,
    out_type=jax.ShapeDtypeStruct((N, D_IN), jnp.float32),
    mesh=_sc_mesh,
    compiler_params=pltpu.CompilerParams(use_tc_tiling_on_sc=False),
    scratch_types=[
        pltpu.VMEM_SHARED((NPAD, DH), jnp.float32),
        pltpu.VMEM_SHARED((NPAD, DH), jnp.float32),
        pltpu.VMEM((64, DH), jnp.float32),
        pltpu.VMEM((DH,), jnp.float32),
        pltpu.VMEM((C,), jnp.int32),
        pltpu.VMEM((C,), jnp.int32),
        pltpu.VMEM((C,), jnp.int32),
        pltpu.VMEM((C,), jnp.int32),
        pltpu.VMEM((C,), jnp.int32),
        pltpu.VMEM((C,), jnp.int32),
        pltpu.VMEM((C,), jnp.int32),
        pltpu.VMEM((C,), jnp.int32),
        pltpu.VMEM((C, DH), jnp.float32),
        pltpu.SemaphoreType.DMA((4,)),
        pltpu.SemaphoreType.DMA,
        pltpu.SemaphoreType.DMA,
    ],
)
def _sc_scatter(h_hbm, ei_hbm, b_hbm, out_hbm,
                h_sh, acc, bias_blk, bvec,
                s0, s1, s2, s3, d0, d1, d2, d3, rv,
                si, sg, ss):
    c = lax.axis_index("c")
    s = lax.axis_index("s")
    srcs = [s0, s1, s2, s3]
    dsts = [d0, d1, d2, d3]

    pltpu.sync_copy(h_hbm.at[c, pl.ds(s * STRIPE, STRIPE)],
                    h_sh.at[pl.ds(s * STRIPE, STRIPE)])

    pltpu.sync_copy(b_hbm.at[pl.ds(c * DH, DH)], bvec)
    vals = [bvec[pl.ds(k * 16, 16)] for k in range(4)]
    for r in range(64):
        for k in range(4):
            bias_blk[r, pl.ds(k * 16, 16)] = vals[k]
    for i in range(STRIPE // 64):
        pltpu.sync_copy(bias_blk, acc.at[pl.ds(s * STRIPE + i * 64, 64)])

    def fire_idx(chunk, buf):
        base = s * EPT + chunk * C
        pltpu.async_copy(ei_hbm.at[0, pl.ds(base, C)], srcs[buf], si.at[buf])
        pltpu.async_copy(ei_hbm.at[1, pl.ds(base, C)], dsts[buf], si.at[buf])

    def wait_idx(buf):
        pltpu.make_async_copy(ei_hbm.at[0, pl.ds(0, C)], srcs[buf],
                              si.at[buf]).wait()
        pltpu.make_async_copy(ei_hbm.at[1, pl.ds(0, C)], dsts[buf],
                              si.at[buf]).wait()

    def step(j, u, first):
        b4 = u % 4
        if not first:
            pltpu.make_async_copy(rv, acc.at[dsts[b4]], ss).wait()
            fire_idx(lax.rem(j + 2, NCHUNK), (u + 2) % 4)
        else:
            fire_idx(j + 2, (u + 2) % 4)
        wait_idx(b4)
        pltpu.async_copy(h_sh.at[srcs[b4]], rv, sg)
        pltpu.make_async_copy(h_sh.at[srcs[b4]], rv, sg).wait()
        pltpu.async_copy(rv, acc.at[dsts[b4]], ss, add=True)

    plsc.subcore_barrier()

    fire_idx(0, 0)
    fire_idx(1, 1)
    step(0, 0, True)
    step(1, 1, False)

    def quad(q, carry):
        j = 4 * q + 2
        step(j + 0, 2, False)
        step(j + 1, 3, False)
        step(j + 2, 0, False)
        step(j + 3, 1, False)
        return carry

    lax.fori_loop(0, (NCHUNK - 2) // 4, quad, 0)

    pltpu.make_async_copy(rv, acc.at[dsts[0]], ss).wait()
    wait_idx(2)
    wait_idx(3)

    plsc.subcore_barrier()

    pltpu.sync_copy(acc.at[pl.ds(s * OSTRIPE, OSTRIPE)],
                    out_hbm.at[pl.ds(s * OSTRIPE, OSTRIPE),
                               pl.ds(c * DH, DH)])


def kernel(x, edge_index, W, b):
    h = _matmul(x, W)
    return _sc_scatter(h, edge_index.astype(jnp.int32), b.astype(jnp.float32))


# speedup vs baseline: 1.2322x; 1.2322x over previous
import functools

import jax
import jax.numpy as jnp
from jax import lax
from jax.experimental import pallas as pl
from jax.experimental.pallas import tpu as pltpu
from jax.experimental.pallas import tpu_sc as plsc

N = 10000
NPAD = 10240
D_IN = 128
DH = 64
E = 320000
C = 200
TILES = 16
EPT = E // TILES
NCHUNK = EPT // C
STRIPE = NPAD // TILES
OSTRIPE = N // TILES


def _mm_body(x_ref, w_ref, o_ref):
    h = jnp.dot(x_ref[...], w_ref[...], preferred_element_type=jnp.float32)
    o_ref[0, :N] = h[:, :DH]
    o_ref[1, :N] = h[:, DH:]


_matmul = pl.pallas_call(
    _mm_body,
    out_shape=jax.ShapeDtypeStruct((2, NPAD, DH), jnp.float32),
)

_sc_mesh = plsc.VectorSubcoreMesh(core_axis_name="c", subcore_axis_name="s")


@functools.partial(
    pl.kernel,
    out_type=jax.ShapeDtypeStruct((N, D_IN), jnp.float32),
    mesh=_sc_mesh,
    compiler_params=pltpu.CompilerParams(use_tc_tiling_on_sc=False),
    scratch_types=[
        pltpu.VMEM_SHARED((NPAD, DH), jnp.float32),
        pltpu.VMEM_SHARED((NPAD, DH), jnp.float32),
        pltpu.VMEM((64, DH), jnp.float32),
        pltpu.VMEM((DH,), jnp.float32),
        pltpu.VMEM((C,), jnp.int32),
        pltpu.VMEM((C,), jnp.int32),
        pltpu.VMEM((C,), jnp.int32),
        pltpu.VMEM((C,), jnp.int32),
        pltpu.VMEM((C,), jnp.int32),
        pltpu.VMEM((C,), jnp.int32),
        pltpu.VMEM((C,), jnp.int32),
        pltpu.VMEM((C,), jnp.int32),
        pltpu.VMEM((C, DH), jnp.float32),
        pltpu.VMEM((C, DH), jnp.float32),
        pltpu.SemaphoreType.DMA((4,)),
        pltpu.SemaphoreType.DMA((2,)),
        pltpu.SemaphoreType.DMA((2,)),
    ],
)
def _sc_scatter(h_hbm, ei_hbm, b_hbm, out_hbm,
                h_sh, acc, bias_blk, bvec,
                s0, s1, s2, s3, d0, d1, d2, d3, r0, r1,
                si, sg, ss):
    c = lax.axis_index("c")
    s = lax.axis_index("s")
    srcs = [s0, s1, s2, s3]
    dsts = [d0, d1, d2, d3]
    rows = [r0, r1]

    pltpu.sync_copy(h_hbm.at[c, pl.ds(s * STRIPE, STRIPE)],
                    h_sh.at[pl.ds(s * STRIPE, STRIPE)])

    pltpu.sync_copy(b_hbm.at[pl.ds(c * DH, DH)], bvec)
    vals = [bvec[pl.ds(k * 16, 16)] for k in range(4)]
    for r in range(64):
        for k in range(4):
            bias_blk[r, pl.ds(k * 16, 16)] = vals[k]
    for i in range(STRIPE // 64):
        pltpu.sync_copy(bias_blk, acc.at[pl.ds(s * STRIPE + i * 64, 64)])

    def fire_idx(chunk, buf):
        base = s * EPT + chunk * C
        pltpu.async_copy(ei_hbm.at[0, pl.ds(base, C)], srcs[buf], si.at[buf])
        pltpu.async_copy(ei_hbm.at[1, pl.ds(base, C)], dsts[buf], si.at[buf])

    def wait_idx(buf):
        pltpu.make_async_copy(ei_hbm.at[0, pl.ds(0, C)], srcs[buf],
                              si.at[buf]).wait()
        pltpu.make_async_copy(ei_hbm.at[1, pl.ds(0, C)], dsts[buf],
                              si.at[buf]).wait()

    def step(j, u, first):
        b2, b4 = u % 2, u % 4
        if not first:
            pltpu.make_async_copy(rows[b2], acc.at[dsts[b4]],
                                  ss.at[b2]).wait()
            fire_idx(lax.rem(j + 2, NCHUNK), (u + 2) % 4)
        else:
            fire_idx(j + 2, (u + 2) % 4)
        wait_idx(b4)
        pltpu.async_copy(h_sh.at[srcs[b4]], rows[b2], sg.at[b2])
        pltpu.make_async_copy(h_sh.at[srcs[b4]], rows[b2], sg.at[b2]).wait()
        pltpu.async_copy(rows[b2], acc.at[dsts[b4]], ss.at[b2], add=True)

    plsc.subcore_barrier()

    fire_idx(0, 0)
    fire_idx(1, 1)
    step(0, 0, True)
    step(1, 1, True)
    step(2, 2, False)
    step(3, 3, False)

    def quad(q, carry):
        j = 4 * q
        step(j + 0, 0, False)
        step(j + 1, 1, False)
        step(j + 2, 2, False)
        step(j + 3, 3, False)
        return carry

    lax.fori_loop(1, NCHUNK // 4, quad, 0)

    pltpu.make_async_copy(rows[0], acc.at[dsts[0]], ss.at[0]).wait()
    pltpu.make_async_copy(rows[1], acc.at[dsts[1]], ss.at[1]).wait()
    wait_idx(0)
    wait_idx(1)

    plsc.subcore_barrier()

    pltpu.sync_copy(acc.at[pl.ds(s * OSTRIPE, OSTRIPE)],
                    out_hbm.at[pl.ds(s * OSTRIPE, OSTRIPE),
                               pl.ds(c * DH, DH)])


def kernel(x, edge_index, W, b):
    h = _matmul(x, W)
    return _sc_scatter(h, edge_index, b)
